# baseline (speedup 1.0000x reference)
import functools

import jax
import jax.numpy as jnp
from jax import lax
from jax.experimental import pallas as pl
from jax.experimental.pallas import tpu as pltpu
from jax.experimental.pallas import tpu_sc as plsc

_TOTAL = 32768


@functools.lru_cache(maxsize=None)
def _make_sc_fill(total: int):
    info = plsc.get_sparse_core_info()
    num_cores, num_subcores, lanes = (
        info.num_cores,
        info.num_subcores,
        info.num_lanes,
    )
    num_workers = num_cores * num_subcores
    per_worker = total // num_workers

    mesh = plsc.VectorSubcoreMesh(core_axis_name="c", subcore_axis_name="s")

    @functools.partial(
        pl.kernel,
        mesh=mesh,
        out_type=jax.ShapeDtypeStruct((total,), jnp.float32),
        scratch_types=[
            pltpu.VMEM((lanes,), jnp.float32),
            pltpu.VMEM((per_worker,), jnp.float32),
        ],
    )
    def fill(param_hbm, out_hbm, param_v, buf_v):
        wid = lax.axis_index("s") * num_cores + lax.axis_index("c")
        pltpu.sync_copy(param_hbm, param_v)
        vec = param_v[...]
        for i in range(per_worker // lanes):
            buf_v[pl.ds(i * lanes, lanes)] = vec
        pltpu.sync_copy(buf_v, out_hbm.at[pl.ds(wid * per_worker, per_worker)])

    return fill


def kernel(repeats, parameter):
    param_flat = jnp.broadcast_to(jnp.reshape(parameter, (1,)), (16,))
    out = _make_sc_fill(_TOTAL)(param_flat)
    return jnp.reshape(out, (_TOTAL, 1))


# speedup vs baseline: 1.0673x; 1.0673x over previous
import functools

import jax
import jax.numpy as jnp
from jax import lax
from jax.experimental import pallas as pl
from jax.experimental.pallas import tpu as pltpu
from jax.experimental.pallas import tpu_sc as plsc

_TOTAL = 32768


@functools.lru_cache(maxsize=None)
def _make_sc_fill(total: int):
    info = plsc.get_sparse_core_info()
    num_cores, num_subcores, lanes = (
        info.num_cores,
        info.num_subcores,
        info.num_lanes,
    )
    num_workers = num_cores * num_subcores
    per_worker = total // num_workers

    num_cores = 1
    num_workers = num_cores * num_subcores
    per_worker = total // num_workers
    mesh = plsc.VectorSubcoreMesh(
        core_axis_name="c", subcore_axis_name="s", num_cores=num_cores
    )

    @functools.partial(
        pl.kernel,
        mesh=mesh,
        out_type=jax.ShapeDtypeStruct((total,), jnp.float32),
        scratch_types=[
            pltpu.VMEM((lanes,), jnp.float32),
            pltpu.VMEM((per_worker,), jnp.float32),
        ],
    )
    def fill(param_hbm, out_hbm, param_v, buf_v):
        wid = lax.axis_index("s") * num_cores + lax.axis_index("c")
        pltpu.sync_copy(param_hbm, param_v)
        vec = param_v[...]
        for i in range(per_worker // lanes):
            buf_v[pl.ds(i * lanes, lanes)] = vec
        pltpu.sync_copy(buf_v, out_hbm.at[pl.ds(wid * per_worker, per_worker)])

    return fill


def kernel(repeats, parameter):
    param_flat = jnp.broadcast_to(jnp.reshape(parameter, (1,)), (16,))
    out = _make_sc_fill(_TOTAL)(param_flat)
    return jnp.reshape(out, (_TOTAL, 1))


# speedup vs baseline: 1.1057x; 1.0360x over previous
import functools

import jax
import jax.numpy as jnp
from jax import lax
from jax.experimental import pallas as pl
from jax.experimental.pallas import tpu as pltpu
from jax.experimental.pallas import tpu_sc as plsc

_TOTAL = 32768


@functools.lru_cache(maxsize=None)
def _make_sc_fill(total: int):
    info = plsc.get_sparse_core_info()
    num_subcores, lanes = info.num_subcores, info.num_lanes
    num_cores = 1
    num_workers = num_cores * num_subcores
    per_worker = total // num_workers

    mesh = plsc.VectorSubcoreMesh(
        core_axis_name="c", subcore_axis_name="s", num_cores=num_cores
    )

    @functools.partial(
        pl.kernel,
        mesh=mesh,
        out_type=jax.ShapeDtypeStruct((total,), jnp.float32),
        scratch_types=[
            pltpu.VMEM((lanes,), jnp.float32),
            pltpu.VMEM((per_worker,), jnp.float32),
        ],
    )
    def fill(param_hbm, out_hbm, param_v, buf_v):
        wid = lax.axis_index("s") * num_cores + lax.axis_index("c")
        pltpu.sync_copy(param_hbm, param_v.at[pl.ds(0, 1)])
        raw = param_v[...]
        vec = lax.gather(
            raw,
            jnp.zeros((lanes, 1), jnp.int32),
            lax.GatherDimensionNumbers(
                offset_dims=(),
                collapsed_slice_dims=(0,),
                start_index_map=(0,),
            ),
            slice_sizes=(1,),
            mode=lax.GatherScatterMode.PROMISE_IN_BOUNDS,
        )
        for i in range(per_worker // lanes):
            buf_v[pl.ds(i * lanes, lanes)] = vec
        pltpu.sync_copy(buf_v, out_hbm.at[pl.ds(wid * per_worker, per_worker)])

    return fill


def kernel(repeats, parameter):
    out = _make_sc_fill(_TOTAL)(jnp.reshape(parameter, (1,)))
    return jnp.reshape(out, (_TOTAL, 1))
